# baseline (speedup 1.0000x reference)
import functools

import numpy as np
import jax
import jax.numpy as jnp
from jax import lax
from jax.experimental import pallas as pl
from jax.experimental.pallas import tpu as pltpu
from jax.experimental.pallas import tpu_sc as plsc

N = 10000
E = 160000
F = 128
DE = 4
DSC = 8
H1 = 64
H2 = 64
FOUT = 128
NUM_NEIGHBORS = 16.0
MIXING_ANGLE = np.pi / 8.0

NC = 2
NS = 16
NW = NC * NS
CHUNK = 128
CPW = 39
CPW_PAD = 40
TRI = 3 * CHUNK
EPW_MAIN = CPW * CHUNK
E_MAIN = NW * EPW_MAIN
REM = (E - E_MAIN) // NW
NBODY = 6
ROWS_PER_SUB = 624
ROWS_TAIL = N - NS * ROWS_PER_SUB

_COS = float(np.cos(MIXING_ANGLE))
_SIN = float(np.sin(MIXING_ANGLE))
_EDGE_SCALE = _SIN / (np.sqrt(H2) * np.sqrt(NUM_NEIGHBORS))

_SC_MESH = plsc.VectorSubcoreMesh(
    core_axis_name="c", subcore_axis_name="s", num_cores=NC, num_subcores=NS
)


def _lin_body(x_ref, w_ref, feat_ref, self_ref):
    t = jnp.dot(x_ref[...], w_ref[...], preferred_element_type=jnp.float32)
    feat_ref[...] = t[:, :F]
    self_ref[...] = t[:, F:] * (0.5 * _COS)


_LIN_ROWS = 2000


def _linear(node_input, W_lin):
    return pl.pallas_call(
        _lin_body,
        grid=(N // _LIN_ROWS,),
        in_specs=[
            pl.BlockSpec((_LIN_ROWS, F), lambda i: (i, 0)),
            pl.BlockSpec((F, F + FOUT), lambda i: (0, 0)),
        ],
        out_specs=[
            pl.BlockSpec((_LIN_ROWS, F), lambda i: (i, 0)),
            pl.BlockSpec((_LIN_ROWS, FOUT), lambda i: (i, 0)),
        ],
        out_shape=[
            jax.ShapeDtypeStruct((N, F), jnp.float32),
            jax.ShapeDtypeStruct((N, FOUT), jnp.float32),
        ],
    )(node_input, W_lin)


@functools.partial(
    pl.kernel,
    out_type=jax.ShapeDtypeStruct((E, F), jnp.float32),
    mesh=_SC_MESH,
    scratch_types=[
        pltpu.VMEM((CPW_PAD, CHUNK), jnp.int32),
        pltpu.VMEM((REM,), jnp.int32),
        pltpu.VMEM((TRI, F), jnp.float32),
        pltpu.VMEM((TRI, F), jnp.float32),
        pltpu.VMEM((REM, F), jnp.float32),
        pltpu.SemaphoreType.DMA,
        pltpu.SemaphoreType.DMA,
        pltpu.SemaphoreType.DMA,
        pltpu.SemaphoreType.DMA,
        pltpu.SemaphoreType.DMA,
    ],
)
def _gather(feat_hbm, src3d_hbm, tail_src_hbm, out_hbm, idx_all, idx_r,
            bufa, bufb, rows_r, sem_ga, sem_gb, sem_oa, sem_ob, sem_t):
    c = lax.axis_index("c")
    s = lax.axis_index("s")
    w = c * NS + s
    base = w * EPW_MAIN

    def fire_gathers(buf, sem, t):
        for b in range(3):
            pltpu.async_copy(feat_hbm.at[idx_all.at[3 * t + b]],
                             buf.at[pl.ds(b * CHUNK, CHUNK)], sem)

    def drain_gathers(buf, sem):
        for b in range(3):
            pltpu.make_async_copy(feat_hbm.at[idx_all.at[0]],
                                  buf.at[pl.ds(b * CHUNK, CHUNK)], sem).wait()

    def fire_out(buf, sem, t):
        pltpu.async_copy(buf, out_hbm.at[pl.ds(base + t * TRI, TRI)], sem)

    def drain_out(buf, sem):
        pltpu.make_async_copy(buf, out_hbm.at[pl.ds(0, TRI)], sem).wait()

    pltpu.sync_copy(src3d_hbm.at[w], idx_all)
    fire_gathers(bufa, sem_ga, 0)

    def body(g, carry):
        @pl.when(g > 0)
        def _():
            drain_out(bufb, sem_ob)
        fire_gathers(bufb, sem_gb, 2 * g + 1)
        drain_gathers(bufa, sem_ga)
        fire_out(bufa, sem_oa, 2 * g)
        drain_gathers(bufb, sem_gb)
        fire_out(bufb, sem_ob, 2 * g + 1)
        drain_out(bufa, sem_oa)
        fire_gathers(bufa, sem_ga, 2 * g + 2)
        return carry

    lax.fori_loop(0, NBODY, body, 0)
    drain_gathers(bufa, sem_ga)
    drain_out(bufb, sem_ob)
    fire_out(bufa, sem_oa, 2 * NBODY)
    offr = E_MAIN + w * REM
    pltpu.sync_copy(tail_src_hbm.at[pl.ds(w * REM, REM)], idx_r)
    pltpu.async_copy(feat_hbm.at[idx_r], rows_r, sem_t).wait()
    pltpu.sync_copy(rows_r, out_hbm.at[pl.ds(offr, REM)])
    drain_out(bufa, sem_oa)


_EB = 2000


def _edge_body(esa_ref, ea_ref, ef_ref, w1_ref, w2_ref, wtp_ref, wout_ref, out_ref):
    h = jax.nn.gelu(jnp.dot(esa_ref[...], w1_ref[...], preferred_element_type=jnp.float32))
    h = jax.nn.gelu(jnp.dot(h, w2_ref[...], preferred_element_type=jnp.float32))
    w_all = jnp.dot(h.astype(jnp.bfloat16), wtp_ref[...],
                    preferred_element_type=jnp.float32)
    ef = ef_ref[...]
    ea = ea_ref[...]
    acc = jnp.zeros((_EB, FOUT), dtype=jnp.float32)
    for j in range(DE):
        mid = w_all[:, j * F:(j + 1) * F] * ef * ea[:, j:j + 1]
        acc = acc + jnp.dot(mid.astype(jnp.bfloat16), wout_ref[j],
                            preferred_element_type=jnp.float32)
    out_ref[...] = acc * _EDGE_SCALE


def _edge_compute(edge_scalar_attr, edge_attr, edge_features, mlp_w1, mlp_w2, wtp2d, wout_perm):
    return pl.pallas_call(
        _edge_body,
        grid=(E // _EB,),
        in_specs=[
            pl.BlockSpec((_EB, DSC), lambda i: (i, 0)),
            pl.BlockSpec((_EB, DE), lambda i: (i, 0)),
            pl.BlockSpec((_EB, F), lambda i: (i, 0)),
            pl.BlockSpec((DSC, H1), lambda i: (0, 0)),
            pl.BlockSpec((H1, H2), lambda i: (0, 0)),
            pl.BlockSpec((H2, DE * F), lambda i: (0, 0)),
            pl.BlockSpec((DE, F, FOUT), lambda i: (0, 0, 0)),
        ],
        out_specs=pl.BlockSpec((_EB, FOUT), lambda i: (i, 0)),
        out_shape=jax.ShapeDtypeStruct((E, FOUT), jnp.float32),
    )(edge_scalar_attr, edge_attr, edge_features, mlp_w1, mlp_w2, wtp2d, wout_perm)


@functools.partial(
    pl.kernel,
    out_type=jax.ShapeDtypeStruct((NC, N, FOUT), jnp.float32),
    mesh=_SC_MESH,
    scratch_types=[
        pltpu.VMEM((CPW_PAD, CHUNK), jnp.int32),
        pltpu.VMEM((REM,), jnp.int32),
        pltpu.VMEM((CHUNK, FOUT), jnp.float32),
        pltpu.VMEM((CHUNK, FOUT), jnp.float32),
        pltpu.VMEM((REM, FOUT), jnp.float32),
        pltpu.VMEM_SHARED((N, FOUT), jnp.float32),
        pltpu.SemaphoreType.DMA,
        pltpu.SemaphoreType.DMA,
        pltpu.SemaphoreType.DMA,
        pltpu.SemaphoreType.DMA,
        pltpu.SemaphoreType.DMA,
    ],
)
def _scatter(edge_out_hbm, dst3d_hbm, tail_dst_hbm, self_hbm, part_hbm,
             idx_all, idx_r, bufa, bufb, rows_r, acc_sh,
             sem_ia, sem_ib, sem_sa, sem_sb, sem_t):
    c = lax.axis_index("c")
    s = lax.axis_index("s")
    w = c * NS + s
    base = w * EPW_MAIN

    def fire_in(buf, sem, i):
        pltpu.async_copy(edge_out_hbm.at[pl.ds(base + i * CHUNK, CHUNK)], buf, sem)

    def drain_in(buf, sem):
        pltpu.make_async_copy(edge_out_hbm.at[pl.ds(0, CHUNK)], buf, sem).wait()

    def fire_scatter(buf, sem, i):
        pltpu.async_copy(buf, acc_sh.at[idx_all.at[i]], sem, add=True)

    def drain_scatter(buf, sem):
        pltpu.make_async_copy(buf, acc_sh.at[idx_all.at[0]], sem).wait()

    r0 = s * ROWS_PER_SUB
    pltpu.sync_copy(self_hbm.at[pl.ds(r0, ROWS_PER_SUB)], acc_sh.at[pl.ds(r0, ROWS_PER_SUB)])
    @pl.when(s == 0)
    def _():
        pltpu.sync_copy(self_hbm.at[pl.ds(NS * ROWS_PER_SUB, ROWS_TAIL)],
                        acc_sh.at[pl.ds(NS * ROWS_PER_SUB, ROWS_TAIL)])
    pltpu.sync_copy(dst3d_hbm.at[w], idx_all)
    plsc.subcore_barrier()
    fire_in(bufa, sem_ia, 0)

    def body(g, carry):
        @pl.when(g > 0)
        def _():
            drain_scatter(bufb, sem_sb)
        fire_in(bufb, sem_ib, 2 * g + 1)
        drain_in(bufa, sem_ia)
        fire_scatter(bufa, sem_sa, 2 * g)
        drain_in(bufb, sem_ib)
        fire_scatter(bufb, sem_sb, 2 * g + 1)
        drain_scatter(bufa, sem_sa)
        fire_in(bufa, sem_ia, 2 * g + 2)
        return carry

    lax.fori_loop(0, (CPW - 1) // 2, body, 0)
    drain_in(bufa, sem_ia)
    fire_scatter(bufa, sem_sa, CPW - 1)
    drain_scatter(bufb, sem_sb)
    offr = E_MAIN + w * REM
    pltpu.sync_copy(tail_dst_hbm.at[pl.ds(w * REM, REM)], idx_r)
    pltpu.sync_copy(edge_out_hbm.at[pl.ds(offr, REM)], rows_r)
    drain_scatter(bufa, sem_sa)
    pltpu.sync_copy(rows_r, acc_sh.at[idx_r], add=True)

    plsc.subcore_barrier()
    pltpu.sync_copy(acc_sh.at[pl.ds(r0, ROWS_PER_SUB)], part_hbm.at[c, pl.ds(r0, ROWS_PER_SUB)])
    @pl.when(s == 0)
    def _():
        pltpu.sync_copy(acc_sh.at[pl.ds(NS * ROWS_PER_SUB, ROWS_TAIL)],
                        part_hbm.at[c, pl.ds(NS * ROWS_PER_SUB, ROWS_TAIL)])


def _combine_body(p0_ref, p1_ref, out_ref):
    out_ref[...] = p0_ref[...] + p1_ref[...]


def _combine(p0, p1):
    return pl.pallas_call(
        _combine_body,
        grid=(N // _LIN_ROWS,),
        in_specs=[
            pl.BlockSpec((_LIN_ROWS, FOUT), lambda i: (i, 0)),
            pl.BlockSpec((_LIN_ROWS, FOUT), lambda i: (i, 0)),
        ],
        out_specs=pl.BlockSpec((_LIN_ROWS, FOUT), lambda i: (i, 0)),
        out_shape=jax.ShapeDtypeStruct((N, FOUT), jnp.float32),
    )(p0, p1)


def _pad_idx_3d(idx):
    main = idx[:E_MAIN].reshape(NW, CPW, CHUNK)
    pad = jnp.zeros((NW, CPW_PAD - CPW, CHUNK), dtype=idx.dtype)
    return jnp.concatenate([main, pad], axis=1)


def kernel(node_input, edge_attr, edge_scalar_attr, W_lin, mlp_w1, mlp_w2, w_tp, W_out, edge_src, edge_dst):
    wtp2d = w_tp.transpose(0, 2, 1).reshape(H2, DE * F).astype(jnp.bfloat16)
    wout_perm = W_out.reshape(F, DE, FOUT).transpose(1, 0, 2).astype(jnp.bfloat16)
    edge_src = edge_src.astype(jnp.int32)
    edge_dst = edge_dst.astype(jnp.int32)
    src3d = _pad_idx_3d(edge_src)
    dst3d = _pad_idx_3d(edge_dst)
    tail_src = edge_src[E_MAIN:]
    tail_dst = edge_dst[E_MAIN:]

    node_features, self_half = _linear(node_input, W_lin)
    edge_features = _gather(node_features, src3d, tail_src)
    edge_out = _edge_compute(edge_scalar_attr, edge_attr, edge_features,
                             mlp_w1, mlp_w2, wtp2d, wout_perm)
    partials = _scatter(edge_out, dst3d, tail_dst, self_half)
    return _combine(partials[0], partials[1])


# speedup vs baseline: 1.1780x; 1.1780x over previous
import functools

import numpy as np
import jax
import jax.numpy as jnp
from jax import lax
from jax.experimental import pallas as pl
from jax.experimental.pallas import tpu as pltpu
from jax.experimental.pallas import tpu_sc as plsc

N = 10000
E = 160000
F = 128
DE = 4
DSC = 8
H1 = 64
H2 = 64
FOUT = 128
NUM_NEIGHBORS = 16.0
MIXING_ANGLE = np.pi / 8.0

NC = 2
NS = 16
NW = NC * NS
CHUNK = 128
CPW = 39
CPW_PAD = 40
TRI = 3 * CHUNK
EPW_MAIN = CPW * CHUNK
E_MAIN = NW * EPW_MAIN
REM = (E - E_MAIN) // NW
NBODY = 6
ROWS_PER_SUB = 624
ROWS_TAIL = N - NS * ROWS_PER_SUB

_COS = float(np.cos(MIXING_ANGLE))
_SIN = float(np.sin(MIXING_ANGLE))
_EDGE_SCALE = _SIN / (np.sqrt(H2) * np.sqrt(NUM_NEIGHBORS))

_SC_MESH = plsc.VectorSubcoreMesh(
    core_axis_name="c", subcore_axis_name="s", num_cores=NC, num_subcores=NS
)


def _lin_body(x_ref, w_ref, feat_ref, self_ref):
    t = jnp.dot(x_ref[...], w_ref[...], preferred_element_type=jnp.float32)
    feat_ref[...] = t[:, :F]
    self_ref[...] = t[:, F:] * (0.5 * _COS)


_LIN_ROWS = 2000


def _linear(node_input, W_lin):
    return pl.pallas_call(
        _lin_body,
        grid=(N // _LIN_ROWS,),
        in_specs=[
            pl.BlockSpec((_LIN_ROWS, F), lambda i: (i, 0)),
            pl.BlockSpec((F, F + FOUT), lambda i: (0, 0)),
        ],
        out_specs=[
            pl.BlockSpec((_LIN_ROWS, F), lambda i: (i, 0)),
            pl.BlockSpec((_LIN_ROWS, FOUT), lambda i: (i, 0)),
        ],
        out_shape=[
            jax.ShapeDtypeStruct((N, F), jnp.float32),
            jax.ShapeDtypeStruct((N, FOUT), jnp.float32),
        ],
    )(node_input, W_lin)


@functools.partial(
    pl.kernel,
    out_type=jax.ShapeDtypeStruct((E, F), jnp.float32),
    mesh=_SC_MESH,
    scratch_types=[
        pltpu.VMEM((CPW_PAD, CHUNK), jnp.int32),
        pltpu.VMEM((REM,), jnp.int32),
        pltpu.VMEM((TRI, F), jnp.float32),
        pltpu.VMEM((TRI, F), jnp.float32),
        pltpu.VMEM((REM, F), jnp.float32),
        pltpu.SemaphoreType.DMA,
        pltpu.SemaphoreType.DMA,
        pltpu.SemaphoreType.DMA,
        pltpu.SemaphoreType.DMA,
        pltpu.SemaphoreType.DMA,
    ],
)
def _gather(feat_hbm, src3d_hbm, tail_src_hbm, out_hbm, idx_all, idx_r,
            bufa, bufb, rows_r, sem_ga, sem_gb, sem_oa, sem_ob, sem_t):
    c = lax.axis_index("c")
    s = lax.axis_index("s")
    w = c * NS + s
    base = w * EPW_MAIN

    def fire_gathers(buf, sem, t):
        for b in range(3):
            pltpu.async_copy(feat_hbm.at[idx_all.at[3 * t + b]],
                             buf.at[pl.ds(b * CHUNK, CHUNK)], sem)

    def drain_gathers(buf, sem):
        for b in range(3):
            pltpu.make_async_copy(feat_hbm.at[idx_all.at[0]],
                                  buf.at[pl.ds(b * CHUNK, CHUNK)], sem).wait()

    def fire_out(buf, sem, t):
        pltpu.async_copy(buf, out_hbm.at[pl.ds(base + t * TRI, TRI)], sem)

    def drain_out(buf, sem):
        pltpu.make_async_copy(buf, out_hbm.at[pl.ds(0, TRI)], sem).wait()

    pltpu.sync_copy(src3d_hbm.at[w], idx_all)
    fire_gathers(bufa, sem_ga, 0)

    def body(g, carry):
        @pl.when(g > 0)
        def _():
            drain_out(bufb, sem_ob)
        fire_gathers(bufb, sem_gb, 2 * g + 1)
        drain_gathers(bufa, sem_ga)
        fire_out(bufa, sem_oa, 2 * g)
        drain_gathers(bufb, sem_gb)
        fire_out(bufb, sem_ob, 2 * g + 1)
        drain_out(bufa, sem_oa)
        fire_gathers(bufa, sem_ga, 2 * g + 2)
        return carry

    lax.fori_loop(0, NBODY, body, 0)
    drain_gathers(bufa, sem_ga)
    drain_out(bufb, sem_ob)
    fire_out(bufa, sem_oa, 2 * NBODY)
    offr = E_MAIN + w * REM
    pltpu.sync_copy(tail_src_hbm.at[pl.ds(w * REM, REM)], idx_r)
    pltpu.async_copy(feat_hbm.at[idx_r], rows_r, sem_t).wait()
    pltpu.sync_copy(rows_r, out_hbm.at[pl.ds(offr, REM)])
    drain_out(bufa, sem_oa)


_EB = 8000


def _edge_body(esa_ref, ea_ref, ef_ref, w1_ref, w2_ref, wtp_ref, wout_ref, out_ref):
    h = jax.nn.gelu(jnp.dot(esa_ref[...], w1_ref[...], preferred_element_type=jnp.float32))
    h = jax.nn.gelu(jnp.dot(h, w2_ref[...], preferred_element_type=jnp.float32))
    w_all = jnp.dot(h, wtp_ref[...], preferred_element_type=jnp.float32)
    ef = ef_ref[...]
    ea = ea_ref[...]
    mid = jnp.concatenate(
        [w_all[:, j * F:(j + 1) * F] * ef * ea[:, j:j + 1] for j in range(DE)],
        axis=1)
    out_ref[...] = jnp.dot(mid, wout_ref[...],
                           preferred_element_type=jnp.float32) * _EDGE_SCALE


def _edge_compute(edge_scalar_attr, edge_attr, edge_features, mlp_w1, mlp_w2, wtp2d, wout_perm):
    return pl.pallas_call(
        _edge_body,
        grid=(E // _EB,),
        in_specs=[
            pl.BlockSpec((_EB, DSC), lambda i: (i, 0)),
            pl.BlockSpec((_EB, DE), lambda i: (i, 0)),
            pl.BlockSpec((_EB, F), lambda i: (i, 0)),
            pl.BlockSpec((DSC, H1), lambda i: (0, 0)),
            pl.BlockSpec((H1, H2), lambda i: (0, 0)),
            pl.BlockSpec((H2, DE * F), lambda i: (0, 0)),
            pl.BlockSpec((DE * F, FOUT), lambda i: (0, 0)),
        ],
        out_specs=pl.BlockSpec((_EB, FOUT), lambda i: (i, 0)),
        out_shape=jax.ShapeDtypeStruct((E, FOUT), jnp.float32),
    )(edge_scalar_attr, edge_attr, edge_features, mlp_w1, mlp_w2, wtp2d, wout_perm)


@functools.partial(
    pl.kernel,
    out_type=jax.ShapeDtypeStruct((NC, N, FOUT), jnp.float32),
    mesh=_SC_MESH,
    scratch_types=[
        pltpu.VMEM((CPW_PAD, CHUNK), jnp.int32),
        pltpu.VMEM((REM,), jnp.int32),
        pltpu.VMEM((CHUNK, FOUT), jnp.float32),
        pltpu.VMEM((CHUNK, FOUT), jnp.float32),
        pltpu.VMEM((REM, FOUT), jnp.float32),
        pltpu.VMEM_SHARED((N, FOUT), jnp.float32),
        pltpu.SemaphoreType.DMA,
        pltpu.SemaphoreType.DMA,
        pltpu.SemaphoreType.DMA,
        pltpu.SemaphoreType.DMA,
        pltpu.SemaphoreType.DMA,
    ],
)
def _scatter(edge_out_hbm, dst3d_hbm, tail_dst_hbm, self_hbm, part_hbm,
             idx_all, idx_r, bufa, bufb, rows_r, acc_sh,
             sem_ia, sem_ib, sem_sa, sem_sb, sem_t):
    c = lax.axis_index("c")
    s = lax.axis_index("s")
    w = c * NS + s
    base = w * EPW_MAIN

    def fire_in(buf, sem, i):
        pltpu.async_copy(edge_out_hbm.at[pl.ds(base + i * CHUNK, CHUNK)], buf, sem)

    def drain_in(buf, sem):
        pltpu.make_async_copy(edge_out_hbm.at[pl.ds(0, CHUNK)], buf, sem).wait()

    def fire_scatter(buf, sem, i):
        pltpu.async_copy(buf, acc_sh.at[idx_all.at[i]], sem, add=True)

    def drain_scatter(buf, sem):
        pltpu.make_async_copy(buf, acc_sh.at[idx_all.at[0]], sem).wait()

    r0 = s * ROWS_PER_SUB
    pltpu.sync_copy(self_hbm.at[pl.ds(r0, ROWS_PER_SUB)], acc_sh.at[pl.ds(r0, ROWS_PER_SUB)])
    @pl.when(s == 0)
    def _():
        pltpu.sync_copy(self_hbm.at[pl.ds(NS * ROWS_PER_SUB, ROWS_TAIL)],
                        acc_sh.at[pl.ds(NS * ROWS_PER_SUB, ROWS_TAIL)])
    pltpu.sync_copy(dst3d_hbm.at[w], idx_all)
    plsc.subcore_barrier()
    fire_in(bufa, sem_ia, 0)

    def body(g, carry):
        @pl.when(g > 0)
        def _():
            drain_scatter(bufb, sem_sb)
        fire_in(bufb, sem_ib, 2 * g + 1)
        drain_in(bufa, sem_ia)
        fire_scatter(bufa, sem_sa, 2 * g)
        drain_in(bufb, sem_ib)
        fire_scatter(bufb, sem_sb, 2 * g + 1)
        drain_scatter(bufa, sem_sa)
        fire_in(bufa, sem_ia, 2 * g + 2)
        return carry

    lax.fori_loop(0, (CPW - 1) // 2, body, 0)
    drain_in(bufa, sem_ia)
    fire_scatter(bufa, sem_sa, CPW - 1)
    drain_scatter(bufb, sem_sb)
    offr = E_MAIN + w * REM
    pltpu.sync_copy(tail_dst_hbm.at[pl.ds(w * REM, REM)], idx_r)
    pltpu.sync_copy(edge_out_hbm.at[pl.ds(offr, REM)], rows_r)
    drain_scatter(bufa, sem_sa)
    pltpu.sync_copy(rows_r, acc_sh.at[idx_r], add=True)

    plsc.subcore_barrier()
    pltpu.sync_copy(acc_sh.at[pl.ds(r0, ROWS_PER_SUB)], part_hbm.at[c, pl.ds(r0, ROWS_PER_SUB)])
    @pl.when(s == 0)
    def _():
        pltpu.sync_copy(acc_sh.at[pl.ds(NS * ROWS_PER_SUB, ROWS_TAIL)],
                        part_hbm.at[c, pl.ds(NS * ROWS_PER_SUB, ROWS_TAIL)])


def _combine_body(p0_ref, p1_ref, out_ref):
    out_ref[...] = p0_ref[...] + p1_ref[...]


def _combine(p0, p1):
    return pl.pallas_call(
        _combine_body,
        grid=(N // _LIN_ROWS,),
        in_specs=[
            pl.BlockSpec((_LIN_ROWS, FOUT), lambda i: (i, 0)),
            pl.BlockSpec((_LIN_ROWS, FOUT), lambda i: (i, 0)),
        ],
        out_specs=pl.BlockSpec((_LIN_ROWS, FOUT), lambda i: (i, 0)),
        out_shape=jax.ShapeDtypeStruct((N, FOUT), jnp.float32),
    )(p0, p1)


def _pad_idx_3d(idx):
    main = idx[:E_MAIN].reshape(NW, CPW, CHUNK)
    pad = jnp.zeros((NW, CPW_PAD - CPW, CHUNK), dtype=idx.dtype)
    return jnp.concatenate([main, pad], axis=1)


def kernel(node_input, edge_attr, edge_scalar_attr, W_lin, mlp_w1, mlp_w2, w_tp, W_out, edge_src, edge_dst):
    wtp2d = w_tp.transpose(0, 2, 1).reshape(H2, DE * F)
    wout_perm = W_out.reshape(F, DE, FOUT).transpose(1, 0, 2).reshape(DE * F, FOUT)
    edge_src = edge_src.astype(jnp.int32)
    edge_dst = edge_dst.astype(jnp.int32)
    src3d = _pad_idx_3d(edge_src)
    dst3d = _pad_idx_3d(edge_dst)
    tail_src = edge_src[E_MAIN:]
    tail_dst = edge_dst[E_MAIN:]

    node_features, self_half = _linear(node_input, W_lin)
    edge_features = _gather(node_features, src3d, tail_src)
    edge_out = _edge_compute(edge_scalar_attr, edge_attr, edge_features,
                             mlp_w1, mlp_w2, wtp2d, wout_perm)
    partials = _scatter(edge_out, dst3d, tail_dst, self_half)
    return _combine(partials[0], partials[1])


# speedup vs baseline: 1.2690x; 1.0772x over previous
import functools

import numpy as np
import jax
import jax.numpy as jnp
from jax import lax
from jax.experimental import pallas as pl
from jax.experimental.pallas import tpu as pltpu
from jax.experimental.pallas import tpu_sc as plsc

N = 10000
E = 160000
F = 128
DE = 4
DSC = 8
H1 = 64
H2 = 64
FOUT = 128
NUM_NEIGHBORS = 16.0
MIXING_ANGLE = np.pi / 8.0

NC = 2
NS = 16
NW = NC * NS
CHUNK = 128
CPW = 39
CPW_PAD = 40
TRI = 3 * CHUNK
EPW_MAIN = CPW * CHUNK
E_MAIN = NW * EPW_MAIN
REM = (E - E_MAIN) // NW
NBODY = 6
ROWS_PER_SUB = 624
ROWS_TAIL = N - NS * ROWS_PER_SUB

_COS = float(np.cos(MIXING_ANGLE))
_SIN = float(np.sin(MIXING_ANGLE))
_EDGE_SCALE = _SIN / (np.sqrt(H2) * np.sqrt(NUM_NEIGHBORS))

_SC_MESH = plsc.VectorSubcoreMesh(
    core_axis_name="c", subcore_axis_name="s", num_cores=NC, num_subcores=NS
)


def _lin_body(x_ref, w_ref, feat_ref, self_ref):
    t = jnp.dot(x_ref[...], w_ref[...], preferred_element_type=jnp.float32)
    feat_ref[...] = t[:, :F]
    self_ref[...] = t[:, F:] * (0.5 * _COS)


_LIN_ROWS = 2000


def _linear(node_input, W_lin):
    return pl.pallas_call(
        _lin_body,
        grid=(N // _LIN_ROWS,),
        in_specs=[
            pl.BlockSpec((_LIN_ROWS, F), lambda i: (i, 0)),
            pl.BlockSpec((F, F + FOUT), lambda i: (0, 0)),
        ],
        out_specs=[
            pl.BlockSpec((_LIN_ROWS, F), lambda i: (i, 0)),
            pl.BlockSpec((_LIN_ROWS, FOUT), lambda i: (i, 0)),
        ],
        out_shape=[
            jax.ShapeDtypeStruct((N, F), jnp.float32),
            jax.ShapeDtypeStruct((N, FOUT), jnp.float32),
        ],
    )(node_input, W_lin)


@functools.partial(
    pl.kernel,
    out_type=jax.ShapeDtypeStruct((E, F), jnp.float32),
    mesh=_SC_MESH,
    scratch_types=[
        pltpu.VMEM((CPW_PAD, CHUNK), jnp.int32),
        pltpu.VMEM((REM,), jnp.int32),
        pltpu.VMEM((TRI, F), jnp.float32),
        pltpu.VMEM((TRI, F), jnp.float32),
        pltpu.VMEM((REM, F), jnp.float32),
        pltpu.SemaphoreType.DMA,
        pltpu.SemaphoreType.DMA,
        pltpu.SemaphoreType.DMA,
        pltpu.SemaphoreType.DMA,
        pltpu.SemaphoreType.DMA,
    ],
)
def _gather(feat_hbm, src3d_hbm, tail_src_hbm, out_hbm, idx_all, idx_r,
            bufa, bufb, rows_r, sem_ga, sem_gb, sem_oa, sem_ob, sem_t):
    c = lax.axis_index("c")
    s = lax.axis_index("s")
    w = c * NS + s
    base = w * EPW_MAIN

    def fire_gathers(buf, sem, t):
        for b in range(3):
            pltpu.async_copy(feat_hbm.at[idx_all.at[3 * t + b]],
                             buf.at[pl.ds(b * CHUNK, CHUNK)], sem)

    def drain_gathers(buf, sem):
        for b in range(3):
            pltpu.make_async_copy(feat_hbm.at[idx_all.at[0]],
                                  buf.at[pl.ds(b * CHUNK, CHUNK)], sem).wait()

    def fire_out(buf, sem, t):
        pltpu.async_copy(buf, out_hbm.at[pl.ds(base + t * TRI, TRI)], sem)

    def drain_out(buf, sem):
        pltpu.make_async_copy(buf, out_hbm.at[pl.ds(0, TRI)], sem).wait()

    pltpu.sync_copy(src3d_hbm.at[w], idx_all)
    fire_gathers(bufa, sem_ga, 0)

    def body(g, carry):
        @pl.when(g > 0)
        def _():
            drain_out(bufb, sem_ob)
        fire_gathers(bufb, sem_gb, 2 * g + 1)
        drain_gathers(bufa, sem_ga)
        fire_out(bufa, sem_oa, 2 * g)
        drain_gathers(bufb, sem_gb)
        fire_out(bufb, sem_ob, 2 * g + 1)
        drain_out(bufa, sem_oa)
        fire_gathers(bufa, sem_ga, 2 * g + 2)
        return carry

    lax.fori_loop(0, NBODY, body, 0)
    drain_gathers(bufa, sem_ga)
    drain_out(bufb, sem_ob)
    fire_out(bufa, sem_oa, 2 * NBODY)
    offr = E_MAIN + w * REM
    pltpu.sync_copy(tail_src_hbm.at[pl.ds(w * REM, REM)], idx_r)
    pltpu.async_copy(feat_hbm.at[idx_r], rows_r, sem_t).wait()
    pltpu.sync_copy(rows_r, out_hbm.at[pl.ds(offr, REM)])
    drain_out(bufa, sem_oa)


_EB = 6400


_DN_T = (((0,), (0,)), ((), ()))


def _edge_body(esa_ref, ea_ref, ef_ref, w1_ref, w2_ref, wtp_ref, wout_ref, out_ref,
               esa_s, ea_s):
    esa_s[...] = lax.dot_general(esa_ref[...], jnp.eye(DSC, dtype=jnp.float32), _DN_T,
                                 preferred_element_type=jnp.float32)
    ea_s[...] = lax.dot_general(ea_ref[...], jnp.eye(DE, dtype=jnp.float32), _DN_T,
                                preferred_element_type=jnp.float32)
    h = jax.nn.gelu(jnp.dot(esa_s[...], w1_ref[...], preferred_element_type=jnp.float32))
    h = jax.nn.gelu(jnp.dot(h, w2_ref[...], preferred_element_type=jnp.float32))
    w_all = jnp.dot(h, wtp_ref[...], preferred_element_type=jnp.float32)
    ea = ea_s[...]
    ef = ef_ref[...]
    mid = jnp.concatenate(
        [w_all[:, j * F:(j + 1) * F] * ef * ea[:, j:j + 1] for j in range(DE)],
        axis=1)
    out_ref[...] = jnp.dot(mid, wout_ref[...],
                           preferred_element_type=jnp.float32) * _EDGE_SCALE


def _edge_compute(edge_scalar_attr, edge_attr, edge_features, mlp_w1, mlp_w2, wtp2d, wout_perm):
    return pl.pallas_call(
        _edge_body,
        grid=(E // _EB,),
        in_specs=[
            pl.BlockSpec((DSC, _EB), lambda i: (0, i)),
            pl.BlockSpec((DE, _EB), lambda i: (0, i)),
            pl.BlockSpec((_EB, F), lambda i: (i, 0)),
            pl.BlockSpec((DSC, H1), lambda i: (0, 0)),
            pl.BlockSpec((H1, H2), lambda i: (0, 0)),
            pl.BlockSpec((H2, DE * F), lambda i: (0, 0)),
            pl.BlockSpec((DE * F, FOUT), lambda i: (0, 0)),
        ],
        out_specs=pl.BlockSpec((_EB, FOUT), lambda i: (i, 0)),
        out_shape=jax.ShapeDtypeStruct((E, FOUT), jnp.float32),
        scratch_shapes=[
            pltpu.VMEM((_EB, DSC), jnp.float32),
            pltpu.VMEM((_EB, DE), jnp.float32),
        ],
    )(edge_scalar_attr, edge_attr, edge_features, mlp_w1, mlp_w2, wtp2d, wout_perm)


@functools.partial(
    pl.kernel,
    out_type=jax.ShapeDtypeStruct((NC, N, FOUT), jnp.float32),
    mesh=_SC_MESH,
    scratch_types=[
        pltpu.VMEM((CPW_PAD, CHUNK), jnp.int32),
        pltpu.VMEM((REM,), jnp.int32),
        pltpu.VMEM((CHUNK, FOUT), jnp.float32),
        pltpu.VMEM((CHUNK, FOUT), jnp.float32),
        pltpu.VMEM((REM, FOUT), jnp.float32),
        pltpu.VMEM_SHARED((N, FOUT), jnp.float32),
        pltpu.SemaphoreType.DMA,
        pltpu.SemaphoreType.DMA,
        pltpu.SemaphoreType.DMA,
        pltpu.SemaphoreType.DMA,
        pltpu.SemaphoreType.DMA,
    ],
)
def _scatter(edge_out_hbm, dst3d_hbm, tail_dst_hbm, self_hbm, part_hbm,
             idx_all, idx_r, bufa, bufb, rows_r, acc_sh,
             sem_ia, sem_ib, sem_sa, sem_sb, sem_t):
    c = lax.axis_index("c")
    s = lax.axis_index("s")
    w = c * NS + s
    base = w * EPW_MAIN

    def fire_in(buf, sem, i):
        pltpu.async_copy(edge_out_hbm.at[pl.ds(base + i * CHUNK, CHUNK)], buf, sem)

    def drain_in(buf, sem):
        pltpu.make_async_copy(edge_out_hbm.at[pl.ds(0, CHUNK)], buf, sem).wait()

    def fire_scatter(buf, sem, i):
        pltpu.async_copy(buf, acc_sh.at[idx_all.at[i]], sem, add=True)

    def drain_scatter(buf, sem):
        pltpu.make_async_copy(buf, acc_sh.at[idx_all.at[0]], sem).wait()

    r0 = s * ROWS_PER_SUB
    pltpu.sync_copy(self_hbm.at[pl.ds(r0, ROWS_PER_SUB)], acc_sh.at[pl.ds(r0, ROWS_PER_SUB)])
    @pl.when(s == 0)
    def _():
        pltpu.sync_copy(self_hbm.at[pl.ds(NS * ROWS_PER_SUB, ROWS_TAIL)],
                        acc_sh.at[pl.ds(NS * ROWS_PER_SUB, ROWS_TAIL)])
    pltpu.sync_copy(dst3d_hbm.at[w], idx_all)
    plsc.subcore_barrier()
    fire_in(bufa, sem_ia, 0)

    def body(g, carry):
        @pl.when(g > 0)
        def _():
            drain_scatter(bufb, sem_sb)
        fire_in(bufb, sem_ib, 2 * g + 1)
        drain_in(bufa, sem_ia)
        fire_scatter(bufa, sem_sa, 2 * g)
        drain_in(bufb, sem_ib)
        fire_scatter(bufb, sem_sb, 2 * g + 1)
        drain_scatter(bufa, sem_sa)
        fire_in(bufa, sem_ia, 2 * g + 2)
        return carry

    lax.fori_loop(0, (CPW - 1) // 2, body, 0)
    drain_in(bufa, sem_ia)
    fire_scatter(bufa, sem_sa, CPW - 1)
    drain_scatter(bufb, sem_sb)
    offr = E_MAIN + w * REM
    pltpu.sync_copy(tail_dst_hbm.at[pl.ds(w * REM, REM)], idx_r)
    pltpu.sync_copy(edge_out_hbm.at[pl.ds(offr, REM)], rows_r)
    drain_scatter(bufa, sem_sa)
    pltpu.sync_copy(rows_r, acc_sh.at[idx_r], add=True)

    plsc.subcore_barrier()
    pltpu.sync_copy(acc_sh.at[pl.ds(r0, ROWS_PER_SUB)], part_hbm.at[c, pl.ds(r0, ROWS_PER_SUB)])
    @pl.when(s == 0)
    def _():
        pltpu.sync_copy(acc_sh.at[pl.ds(NS * ROWS_PER_SUB, ROWS_TAIL)],
                        part_hbm.at[c, pl.ds(NS * ROWS_PER_SUB, ROWS_TAIL)])


def _combine_body(p0_ref, p1_ref, out_ref):
    out_ref[...] = p0_ref[...] + p1_ref[...]


def _combine(p0, p1):
    return pl.pallas_call(
        _combine_body,
        grid=(N // _LIN_ROWS,),
        in_specs=[
            pl.BlockSpec((_LIN_ROWS, FOUT), lambda i: (i, 0)),
            pl.BlockSpec((_LIN_ROWS, FOUT), lambda i: (i, 0)),
        ],
        out_specs=pl.BlockSpec((_LIN_ROWS, FOUT), lambda i: (i, 0)),
        out_shape=jax.ShapeDtypeStruct((N, FOUT), jnp.float32),
    )(p0, p1)


def _pad_idx_3d(idx):
    main = idx[:E_MAIN].reshape(NW, CPW, CHUNK)
    pad = jnp.zeros((NW, CPW_PAD - CPW, CHUNK), dtype=idx.dtype)
    return jnp.concatenate([main, pad], axis=1)


def kernel(node_input, edge_attr, edge_scalar_attr, W_lin, mlp_w1, mlp_w2, w_tp, W_out, edge_src, edge_dst):
    wtp2d = w_tp.transpose(0, 2, 1).reshape(H2, DE * F)
    wout_perm = W_out.reshape(F, DE, FOUT).transpose(1, 0, 2).reshape(DE * F, FOUT)
    edge_src = edge_src.astype(jnp.int32)
    edge_dst = edge_dst.astype(jnp.int32)
    src3d = _pad_idx_3d(edge_src)
    dst3d = _pad_idx_3d(edge_dst)
    tail_src = edge_src[E_MAIN:]
    tail_dst = edge_dst[E_MAIN:]

    esa_t = edge_scalar_attr.T
    ea_t = edge_attr.T

    node_features, self_half = _linear(node_input, W_lin)
    edge_features = _gather(node_features, src3d, tail_src)
    edge_out = _edge_compute(esa_t, ea_t, edge_features,
                             mlp_w1, mlp_w2, wtp2d, wout_perm)
    partials = _scatter(edge_out, dst3d, tail_dst, self_half)
    return _combine(partials[0], partials[1])


# speedup vs baseline: 1.5010x; 1.1828x over previous
import functools

import numpy as np
import jax
import jax.numpy as jnp
from jax import lax
from jax.experimental import pallas as pl
from jax.experimental.pallas import tpu as pltpu
from jax.experimental.pallas import tpu_sc as plsc

N = 10000
E = 160000
F = 128
DE = 4
DSC = 8
H1 = 64
H2 = 64
FOUT = 128
NUM_NEIGHBORS = 16.0
MIXING_ANGLE = np.pi / 8.0

NC = 2
NS = 16
NW = NC * NS
CHUNK = 128
E_PAD = 163840
NPH = 2
EPH = E_PAD // NPH
CPW = 20
CPW_PAD = 24
EPW = CPW * CHUNK
PAIR = 2 * CHUNK
NPAIR = CPW // 2
ROWS_PER_SUB = 624
ROWS_TAIL = N - NS * ROWS_PER_SUB

_COS = float(np.cos(MIXING_ANGLE))
_SIN = float(np.sin(MIXING_ANGLE))
_EDGE_SCALE = _SIN / (np.sqrt(H2) * np.sqrt(NUM_NEIGHBORS))

_SC_MESH = plsc.VectorSubcoreMesh(
    core_axis_name="c", subcore_axis_name="s", num_cores=NC, num_subcores=NS
)


def _lin_body(x_ref, w_ref, feat_ref, self_ref):
    t = jnp.dot(x_ref[...], w_ref[...], preferred_element_type=jnp.float32)
    feat_ref[...] = t[:, :F]
    self_ref[...] = t[:, F:] * (0.25 * _COS)


_LIN_ROWS = 2000


def _linear(node_input, W_lin):
    return pl.pallas_call(
        _lin_body,
        grid=(N // _LIN_ROWS,),
        in_specs=[
            pl.BlockSpec((_LIN_ROWS, F), lambda i: (i, 0)),
            pl.BlockSpec((F, F + FOUT), lambda i: (0, 0)),
        ],
        out_specs=[
            pl.BlockSpec((_LIN_ROWS, F), lambda i: (i, 0)),
            pl.BlockSpec((_LIN_ROWS, FOUT), lambda i: (i, 0)),
        ],
        out_shape=[
            jax.ShapeDtypeStruct((N, F), jnp.float32),
            jax.ShapeDtypeStruct((N, FOUT), jnp.float32),
        ],
    )(node_input, W_lin)


@functools.partial(
    pl.kernel,
    out_type=jax.ShapeDtypeStruct((EPH, F), jnp.float32),
    mesh=_SC_MESH,
    scratch_types=[
        pltpu.VMEM((CPW_PAD, CHUNK), jnp.int32),
        pltpu.VMEM((PAIR, F), jnp.float32),
        pltpu.VMEM((PAIR, F), jnp.float32),
        pltpu.SemaphoreType.DMA,
        pltpu.SemaphoreType.DMA,
        pltpu.SemaphoreType.DMA,
        pltpu.SemaphoreType.DMA,
    ],
)
def _gather(feat_hbm, src3d_hbm, out_hbm, idx_all, bufa, bufb,
            sem_ga, sem_gb, sem_oa, sem_ob):
    c = lax.axis_index("c")
    s = lax.axis_index("s")
    w = c * NS + s
    base = w * EPW

    def fire_gathers(buf, sem, u):
        for b in range(2):
            pltpu.async_copy(feat_hbm.at[idx_all.at[2 * u + b]],
                             buf.at[pl.ds(b * CHUNK, CHUNK)], sem)

    def drain_gathers(buf, sem):
        for b in range(2):
            pltpu.make_async_copy(feat_hbm.at[idx_all.at[0]],
                                  buf.at[pl.ds(b * CHUNK, CHUNK)], sem).wait()

    def fire_out(buf, sem, u):
        pltpu.async_copy(buf, out_hbm.at[pl.ds(base + u * PAIR, PAIR)], sem)

    def drain_out(buf, sem):
        pltpu.make_async_copy(buf, out_hbm.at[pl.ds(0, PAIR)], sem).wait()

    pltpu.sync_copy(src3d_hbm.at[w], idx_all)
    fire_gathers(bufa, sem_ga, 0)

    def body(g, carry):
        @pl.when(g > 0)
        def _():
            drain_out(bufb, sem_ob)
        fire_gathers(bufb, sem_gb, 2 * g + 1)
        drain_gathers(bufa, sem_ga)
        fire_out(bufa, sem_oa, 2 * g)
        drain_gathers(bufb, sem_gb)
        fire_out(bufb, sem_ob, 2 * g + 1)
        drain_out(bufa, sem_oa)
        fire_gathers(bufa, sem_ga, 2 * g + 2)
        return carry

    lax.fori_loop(0, NPAIR // 2 - 1, body, 0)
    gl = NPAIR // 2 - 1
    drain_out(bufb, sem_ob)
    fire_gathers(bufb, sem_gb, 2 * gl + 1)
    drain_gathers(bufa, sem_ga)
    fire_out(bufa, sem_oa, 2 * gl)
    drain_gathers(bufb, sem_gb)
    fire_out(bufb, sem_ob, 2 * gl + 1)
    drain_out(bufa, sem_oa)
    drain_out(bufb, sem_ob)


_EB = 8192

_DN_T = (((0,), (0,)), ((), ()))


def _edge_body(esa_ref, ea_ref, ef_ref, w1_ref, w2_ref, wtp_ref, wout_ref, out_ref,
               esa_s, ea_s):
    esa_s[...] = lax.dot_general(esa_ref[...], jnp.eye(DSC, dtype=jnp.float32), _DN_T,
                                 preferred_element_type=jnp.float32)
    ea_s[...] = lax.dot_general(ea_ref[...], jnp.eye(DE, dtype=jnp.float32), _DN_T,
                                preferred_element_type=jnp.float32)
    h = jax.nn.gelu(jnp.dot(esa_s[...], w1_ref[...], preferred_element_type=jnp.float32))
    h = jax.nn.gelu(jnp.dot(h, w2_ref[...], preferred_element_type=jnp.float32))
    w_all = jnp.dot(h, wtp_ref[...], preferred_element_type=jnp.float32)
    ea = ea_s[...]
    ef = ef_ref[...]
    mid = jnp.concatenate(
        [w_all[:, j * F:(j + 1) * F] * ef * ea[:, j:j + 1] for j in range(DE)],
        axis=1)
    out_ref[...] = jnp.dot(mid, wout_ref[...],
                           preferred_element_type=jnp.float32) * _EDGE_SCALE


def _edge_compute(phase, esa_t, ea_t, edge_features, mlp_w1, mlp_w2, wtp2d, wout_perm):
    nb = EPH // _EB
    off = phase * nb
    return pl.pallas_call(
        _edge_body,
        grid=(nb,),
        in_specs=[
            pl.BlockSpec((DSC, _EB), lambda i: (0, i + off)),
            pl.BlockSpec((DE, _EB), lambda i: (0, i + off)),
            pl.BlockSpec((_EB, F), lambda i: (i, 0)),
            pl.BlockSpec((DSC, H1), lambda i: (0, 0)),
            pl.BlockSpec((H1, H2), lambda i: (0, 0)),
            pl.BlockSpec((H2, DE * F), lambda i: (0, 0)),
            pl.BlockSpec((DE * F, FOUT), lambda i: (0, 0)),
        ],
        out_specs=pl.BlockSpec((_EB, FOUT), lambda i: (i, 0)),
        out_shape=jax.ShapeDtypeStruct((EPH, FOUT), jnp.float32),
        scratch_shapes=[
            pltpu.VMEM((_EB, DSC), jnp.float32),
            pltpu.VMEM((_EB, DE), jnp.float32),
        ],
    )(esa_t, ea_t, edge_features, mlp_w1, mlp_w2, wtp2d, wout_perm)


@functools.partial(
    pl.kernel,
    out_type=jax.ShapeDtypeStruct((NC, N, FOUT), jnp.float32),
    mesh=_SC_MESH,
    scratch_types=[
        pltpu.VMEM((CPW_PAD, CHUNK), jnp.int32),
        pltpu.VMEM((CHUNK, FOUT), jnp.float32),
        pltpu.VMEM((CHUNK, FOUT), jnp.float32),
        pltpu.VMEM_SHARED((N, FOUT), jnp.float32),
        pltpu.SemaphoreType.DMA,
        pltpu.SemaphoreType.DMA,
        pltpu.SemaphoreType.DMA,
        pltpu.SemaphoreType.DMA,
    ],
)
def _scatter(edge_out_hbm, dst3d_hbm, self_hbm, part_hbm,
             idx_all, bufa, bufb, acc_sh, sem_ia, sem_ib, sem_sa, sem_sb):
    c = lax.axis_index("c")
    s = lax.axis_index("s")
    w = c * NS + s
    base = w * EPW

    def fire_in(buf, sem, i):
        pltpu.async_copy(edge_out_hbm.at[pl.ds(base + i * CHUNK, CHUNK)], buf, sem)

    def drain_in(buf, sem):
        pltpu.make_async_copy(edge_out_hbm.at[pl.ds(0, CHUNK)], buf, sem).wait()

    def fire_scatter(buf, sem, i):
        pltpu.async_copy(buf, acc_sh.at[idx_all.at[i]], sem, add=True)

    def drain_scatter(buf, sem):
        pltpu.make_async_copy(buf, acc_sh.at[idx_all.at[0]], sem).wait()

    r0 = s * ROWS_PER_SUB
    pltpu.sync_copy(self_hbm.at[pl.ds(r0, ROWS_PER_SUB)], acc_sh.at[pl.ds(r0, ROWS_PER_SUB)])
    @pl.when(s == 0)
    def _():
        pltpu.sync_copy(self_hbm.at[pl.ds(NS * ROWS_PER_SUB, ROWS_TAIL)],
                        acc_sh.at[pl.ds(NS * ROWS_PER_SUB, ROWS_TAIL)])
    pltpu.sync_copy(dst3d_hbm.at[w], idx_all)
    plsc.subcore_barrier()
    fire_in(bufa, sem_ia, 0)

    def body(g, carry):
        @pl.when(g > 0)
        def _():
            drain_scatter(bufb, sem_sb)
        fire_in(bufb, sem_ib, 2 * g + 1)
        drain_in(bufa, sem_ia)
        fire_scatter(bufa, sem_sa, 2 * g)
        drain_in(bufb, sem_ib)
        fire_scatter(bufb, sem_sb, 2 * g + 1)
        drain_scatter(bufa, sem_sa)
        fire_in(bufa, sem_ia, 2 * g + 2)
        return carry

    lax.fori_loop(0, CPW // 2 - 1, body, 0)
    gl = CPW // 2 - 1
    drain_scatter(bufb, sem_sb)
    fire_in(bufb, sem_ib, 2 * gl + 1)
    drain_in(bufa, sem_ia)
    fire_scatter(bufa, sem_sa, 2 * gl)
    drain_in(bufb, sem_ib)
    fire_scatter(bufb, sem_sb, 2 * gl + 1)
    drain_scatter(bufa, sem_sa)
    drain_scatter(bufb, sem_sb)

    plsc.subcore_barrier()
    pltpu.sync_copy(acc_sh.at[pl.ds(r0, ROWS_PER_SUB)], part_hbm.at[c, pl.ds(r0, ROWS_PER_SUB)])
    @pl.when(s == 0)
    def _():
        pltpu.sync_copy(acc_sh.at[pl.ds(NS * ROWS_PER_SUB, ROWS_TAIL)],
                        part_hbm.at[c, pl.ds(NS * ROWS_PER_SUB, ROWS_TAIL)])


def _combine_body(pa_ref, pb_ref, out_ref):
    out_ref[...] = (pa_ref[0] + pa_ref[1]) + (pb_ref[0] + pb_ref[1])


def _combine(pa, pb):
    return pl.pallas_call(
        _combine_body,
        grid=(N // _LIN_ROWS,),
        in_specs=[
            pl.BlockSpec((NC, _LIN_ROWS, FOUT), lambda i: (0, i, 0)),
            pl.BlockSpec((NC, _LIN_ROWS, FOUT), lambda i: (0, i, 0)),
        ],
        out_specs=pl.BlockSpec((_LIN_ROWS, FOUT), lambda i: (i, 0)),
        out_shape=jax.ShapeDtypeStruct((N, FOUT), jnp.float32),
    )(pa, pb)


def kernel(node_input, edge_attr, edge_scalar_attr, W_lin, mlp_w1, mlp_w2, w_tp, W_out, edge_src, edge_dst):
    wtp2d = w_tp.transpose(0, 2, 1).reshape(H2, DE * F)
    wout_perm = W_out.reshape(F, DE, FOUT).transpose(1, 0, 2).reshape(DE * F, FOUT)
    npad = E_PAD - E
    pad_idx = (jnp.arange(npad, dtype=jnp.int32) * 37) % N
    edge_src = jnp.concatenate([edge_src.astype(jnp.int32), pad_idx])
    edge_dst = jnp.concatenate([edge_dst.astype(jnp.int32), pad_idx])
    esa_t = jnp.concatenate(
        [edge_scalar_attr, jnp.zeros((npad, DSC), jnp.float32)]).T
    ea_t = jnp.concatenate(
        [edge_attr, jnp.zeros((npad, DE), jnp.float32)]).T

    def idx3d(idx):
        main = idx.reshape(NPH, NW, CPW, CHUNK)
        pad = jnp.zeros((NPH, NW, CPW_PAD - CPW, CHUNK), dtype=idx.dtype)
        return jnp.concatenate([main, pad], axis=2)

    src3d = idx3d(edge_src)
    dst3d = idx3d(edge_dst)

    node_features, self_q = _linear(node_input, W_lin)
    parts = []
    for p in range(NPH):
        ef_p = _gather(node_features, src3d[p])
        eo_p = _edge_compute(p, esa_t, ea_t, ef_p, mlp_w1, mlp_w2, wtp2d, wout_perm)
        parts.append(_scatter(eo_p, dst3d[p], self_q))
    return _combine(parts[0], parts[1])


# speedup vs baseline: 1.5172x; 1.0108x over previous
import functools

import numpy as np
import jax
import jax.numpy as jnp
from jax import lax
from jax.experimental import pallas as pl
from jax.experimental.pallas import tpu as pltpu
from jax.experimental.pallas import tpu_sc as plsc

N = 10000
E = 160000
F = 128
DE = 4
DSC = 8
H1 = 64
H2 = 64
FOUT = 128
NUM_NEIGHBORS = 16.0
MIXING_ANGLE = np.pi / 8.0

NC = 2
NS = 16
NW = NC * NS
CHUNK = 128
E_PAD = 163840
NPH = 2
EPH = E_PAD // NPH
CPW = 20
CPW_PAD = 24
EPW = CPW * CHUNK
PAIR = 2 * CHUNK
NPAIR = CPW // 2
ROWS_PER_SUB = 624
ROWS_TAIL = N - NS * ROWS_PER_SUB

_COS = float(np.cos(MIXING_ANGLE))
_SIN = float(np.sin(MIXING_ANGLE))
_EDGE_SCALE = _SIN / (np.sqrt(H2) * np.sqrt(NUM_NEIGHBORS))

_SC_MESH = plsc.VectorSubcoreMesh(
    core_axis_name="c", subcore_axis_name="s", num_cores=NC, num_subcores=NS
)


def _lin_body(x_ref, w_ref, feat_ref, self_ref):
    t = jnp.dot(x_ref[...], w_ref[...], preferred_element_type=jnp.float32)
    feat_ref[...] = t[:, :F]
    self_ref[...] = t[:, F:] * (0.25 * _COS)


_LIN_ROWS = 2000


def _linear(node_input, W_lin):
    return pl.pallas_call(
        _lin_body,
        grid=(N // _LIN_ROWS,),
        in_specs=[
            pl.BlockSpec((_LIN_ROWS, F), lambda i: (i, 0)),
            pl.BlockSpec((F, F + FOUT), lambda i: (0, 0)),
        ],
        out_specs=[
            pl.BlockSpec((_LIN_ROWS, F), lambda i: (i, 0)),
            pl.BlockSpec((_LIN_ROWS, FOUT), lambda i: (i, 0)),
        ],
        out_shape=[
            jax.ShapeDtypeStruct((N, F), jnp.float32),
            jax.ShapeDtypeStruct((N, FOUT), jnp.float32),
        ],
    )(node_input, W_lin)


@functools.partial(
    pl.kernel,
    out_type=jax.ShapeDtypeStruct((EPH, F), jnp.float32),
    mesh=_SC_MESH,
    scratch_types=[
        pltpu.VMEM((CPW_PAD, CHUNK), jnp.int32),
        pltpu.VMEM((PAIR, F), jnp.float32),
        pltpu.VMEM((PAIR, F), jnp.float32),
        pltpu.SemaphoreType.DMA,
        pltpu.SemaphoreType.DMA,
        pltpu.SemaphoreType.DMA,
        pltpu.SemaphoreType.DMA,
    ],
)
def _gather(feat_hbm, src3d_hbm, out_hbm, idx_all, bufa, bufb,
            sem_ga, sem_gb, sem_oa, sem_ob):
    c = lax.axis_index("c")
    s = lax.axis_index("s")
    w = c * NS + s
    base = w * EPW

    def fire_gathers(buf, sem, u):
        for b in range(2):
            pltpu.async_copy(feat_hbm.at[idx_all.at[2 * u + b]],
                             buf.at[pl.ds(b * CHUNK, CHUNK)], sem)

    def drain_gathers(buf, sem):
        for b in range(2):
            pltpu.make_async_copy(feat_hbm.at[idx_all.at[0]],
                                  buf.at[pl.ds(b * CHUNK, CHUNK)], sem).wait()

    def fire_out(buf, sem, u):
        pltpu.async_copy(buf, out_hbm.at[pl.ds(base + u * PAIR, PAIR)], sem)

    def drain_out(buf, sem):
        pltpu.make_async_copy(buf, out_hbm.at[pl.ds(0, PAIR)], sem).wait()

    pltpu.sync_copy(src3d_hbm.at[w], idx_all)
    fire_gathers(bufa, sem_ga, 0)

    def body(g, carry):
        @pl.when(g > 0)
        def _():
            drain_out(bufb, sem_ob)
        fire_gathers(bufb, sem_gb, 2 * g + 1)
        drain_gathers(bufa, sem_ga)
        fire_out(bufa, sem_oa, 2 * g)
        drain_gathers(bufb, sem_gb)
        fire_out(bufb, sem_ob, 2 * g + 1)
        drain_out(bufa, sem_oa)
        fire_gathers(bufa, sem_ga, 2 * g + 2)
        return carry

    lax.fori_loop(0, NPAIR // 2 - 1, body, 0)
    gl = NPAIR // 2 - 1
    drain_out(bufb, sem_ob)
    fire_gathers(bufb, sem_gb, 2 * gl + 1)
    drain_gathers(bufa, sem_ga)
    fire_out(bufa, sem_oa, 2 * gl)
    drain_gathers(bufb, sem_gb)
    fire_out(bufb, sem_ob, 2 * gl + 1)
    drain_out(bufa, sem_oa)
    drain_out(bufb, sem_ob)


_EB = 8192

_DN_T = (((0,), (0,)), ((), ()))


def _edge_body(attr_ref, ef_ref, w1_ref, w2_ref, wtp_ref, wout_ref, out_ref):
    attr = attr_ref[...]
    h1t = jax.nn.gelu(lax.dot_general(w1_ref[...], attr, _DN_T,
                                      preferred_element_type=jnp.float32))
    h2t = jax.nn.gelu(lax.dot_general(w2_ref[...], h1t, _DN_T,
                                      preferred_element_type=jnp.float32))
    ef = ef_ref[...]
    mid = jnp.concatenate(
        [lax.dot_general(h2t * attr[DSC + j:DSC + j + 1, :],
                         wtp_ref[:, j * F:(j + 1) * F], _DN_T,
                         preferred_element_type=jnp.float32) * ef
         for j in range(DE)],
        axis=1)
    out_ref[...] = jnp.dot(mid, wout_ref[...],
                           preferred_element_type=jnp.float32) * _EDGE_SCALE


def _edge_compute(phase, attr_t, edge_features, mlp_w1, mlp_w2, wtp2d, wout_perm):
    nb = EPH // _EB
    off = phase * nb
    return pl.pallas_call(
        _edge_body,
        grid=(nb,),
        in_specs=[
            pl.BlockSpec((DSC + DE, _EB), lambda i: (0, i + off)),
            pl.BlockSpec((_EB, F), lambda i: (i, 0)),
            pl.BlockSpec((DSC + DE, H1), lambda i: (0, 0)),
            pl.BlockSpec((H1, H2), lambda i: (0, 0)),
            pl.BlockSpec((H2, DE * F), lambda i: (0, 0)),
            pl.BlockSpec((DE * F, FOUT), lambda i: (0, 0)),
        ],
        out_specs=pl.BlockSpec((_EB, FOUT), lambda i: (i, 0)),
        out_shape=jax.ShapeDtypeStruct((EPH, FOUT), jnp.float32),
    )(attr_t, edge_features, mlp_w1, mlp_w2, wtp2d, wout_perm)


@functools.partial(
    pl.kernel,
    out_type=jax.ShapeDtypeStruct((NC, N, FOUT), jnp.float32),
    mesh=_SC_MESH,
    scratch_types=[
        pltpu.VMEM((CPW_PAD, CHUNK), jnp.int32),
        pltpu.VMEM((CHUNK, FOUT), jnp.float32),
        pltpu.VMEM((CHUNK, FOUT), jnp.float32),
        pltpu.VMEM_SHARED((N, FOUT), jnp.float32),
        pltpu.SemaphoreType.DMA,
        pltpu.SemaphoreType.DMA,
        pltpu.SemaphoreType.DMA,
        pltpu.SemaphoreType.DMA,
    ],
)
def _scatter(edge_out_hbm, dst3d_hbm, self_hbm, part_hbm,
             idx_all, bufa, bufb, acc_sh, sem_ia, sem_ib, sem_sa, sem_sb):
    c = lax.axis_index("c")
    s = lax.axis_index("s")
    w = c * NS + s
    base = w * EPW

    def fire_in(buf, sem, i):
        pltpu.async_copy(edge_out_hbm.at[pl.ds(base + i * CHUNK, CHUNK)], buf, sem)

    def drain_in(buf, sem):
        pltpu.make_async_copy(edge_out_hbm.at[pl.ds(0, CHUNK)], buf, sem).wait()

    def fire_scatter(buf, sem, i):
        pltpu.async_copy(buf, acc_sh.at[idx_all.at[i]], sem, add=True)

    def drain_scatter(buf, sem):
        pltpu.make_async_copy(buf, acc_sh.at[idx_all.at[0]], sem).wait()

    r0 = s * ROWS_PER_SUB
    pltpu.sync_copy(self_hbm.at[pl.ds(r0, ROWS_PER_SUB)], acc_sh.at[pl.ds(r0, ROWS_PER_SUB)])
    @pl.when(s == 0)
    def _():
        pltpu.sync_copy(self_hbm.at[pl.ds(NS * ROWS_PER_SUB, ROWS_TAIL)],
                        acc_sh.at[pl.ds(NS * ROWS_PER_SUB, ROWS_TAIL)])
    pltpu.sync_copy(dst3d_hbm.at[w], idx_all)
    plsc.subcore_barrier()
    fire_in(bufa, sem_ia, 0)

    def body(g, carry):
        @pl.when(g > 0)
        def _():
            drain_scatter(bufb, sem_sb)
        fire_in(bufb, sem_ib, 2 * g + 1)
        drain_in(bufa, sem_ia)
        fire_scatter(bufa, sem_sa, 2 * g)
        drain_in(bufb, sem_ib)
        fire_scatter(bufb, sem_sb, 2 * g + 1)
        drain_scatter(bufa, sem_sa)
        fire_in(bufa, sem_ia, 2 * g + 2)
        return carry

    lax.fori_loop(0, CPW // 2 - 1, body, 0)
    gl = CPW // 2 - 1
    drain_scatter(bufb, sem_sb)
    fire_in(bufb, sem_ib, 2 * gl + 1)
    drain_in(bufa, sem_ia)
    fire_scatter(bufa, sem_sa, 2 * gl)
    drain_in(bufb, sem_ib)
    fire_scatter(bufb, sem_sb, 2 * gl + 1)
    drain_scatter(bufa, sem_sa)
    drain_scatter(bufb, sem_sb)

    plsc.subcore_barrier()
    pltpu.sync_copy(acc_sh.at[pl.ds(r0, ROWS_PER_SUB)], part_hbm.at[c, pl.ds(r0, ROWS_PER_SUB)])
    @pl.when(s == 0)
    def _():
        pltpu.sync_copy(acc_sh.at[pl.ds(NS * ROWS_PER_SUB, ROWS_TAIL)],
                        part_hbm.at[c, pl.ds(NS * ROWS_PER_SUB, ROWS_TAIL)])


def _combine_body(pa_ref, pb_ref, out_ref):
    out_ref[...] = (pa_ref[0] + pa_ref[1]) + (pb_ref[0] + pb_ref[1])


def _combine(pa, pb):
    return pl.pallas_call(
        _combine_body,
        grid=(N // _LIN_ROWS,),
        in_specs=[
            pl.BlockSpec((NC, _LIN_ROWS, FOUT), lambda i: (0, i, 0)),
            pl.BlockSpec((NC, _LIN_ROWS, FOUT), lambda i: (0, i, 0)),
        ],
        out_specs=pl.BlockSpec((_LIN_ROWS, FOUT), lambda i: (i, 0)),
        out_shape=jax.ShapeDtypeStruct((N, FOUT), jnp.float32),
    )(pa, pb)


def kernel(node_input, edge_attr, edge_scalar_attr, W_lin, mlp_w1, mlp_w2, w_tp, W_out, edge_src, edge_dst):
    wtp2d = w_tp.transpose(0, 2, 1).reshape(H2, DE * F)
    wout_perm = W_out.reshape(F, DE, FOUT).transpose(1, 0, 2).reshape(DE * F, FOUT)
    mlp_w1 = jnp.concatenate([mlp_w1, jnp.zeros((DE, H1), jnp.float32)])
    npad = E_PAD - E
    pad_idx = (jnp.arange(npad, dtype=jnp.int32) * 37) % N
    edge_src = jnp.concatenate([edge_src.astype(jnp.int32), pad_idx])
    edge_dst = jnp.concatenate([edge_dst.astype(jnp.int32), pad_idx])
    attr = jnp.concatenate(
        [jnp.concatenate([edge_scalar_attr, edge_attr], axis=1),
         jnp.zeros((npad, DSC + DE), jnp.float32)])
    attr_t = attr.T

    def idx3d(idx):
        main = idx.reshape(NPH, NW, CPW, CHUNK)
        pad = jnp.zeros((NPH, NW, CPW_PAD - CPW, CHUNK), dtype=idx.dtype)
        return jnp.concatenate([main, pad], axis=2)

    src3d = idx3d(edge_src)
    dst3d = idx3d(edge_dst)

    node_features, self_q = _linear(node_input, W_lin)
    parts = []
    for p in range(NPH):
        ef_p = _gather(node_features, src3d[p])
        eo_p = _edge_compute(p, attr_t, ef_p, mlp_w1, mlp_w2, wtp2d, wout_perm)
        parts.append(_scatter(eo_p, dst3d[p], self_q))
    return _combine(parts[0], parts[1])
